# baseline (speedup 1.0000x reference)
import functools

import jax
import jax.numpy as jnp
from jax import lax
from jax.experimental import pallas as pl
from jax.experimental.pallas import tpu as pltpu
from jax.experimental.pallas import tpu_sc as plsc

B = 16384
FIELD = 26
EMB = 16
V = 26 * 40000
D = FIELD * EMB
DH = D // 2
NUMF = 13
EPS = 1e-5

NC, NS = 2, 16
NW = NC * NS
N_IDX = B * FIELD
PER_W = N_IDX // NW
K = 128
NSTREAM = PER_W // K

NB = 4
MB = NB * K
NMACRO = NSTREAM // NB
SAMP_W = B // NW

TCOL = V // 128
CTC = 16
NCHUNK = (TCOL + CTC - 1) // CTC
NROW8 = NCHUNK * 2048

_sc_mesh = plsc.VectorSubcoreMesh(core_axis_name="c", subcore_axis_name="s")


def _trans_body(in_ref, out_ref):
    x = in_ref[...]
    y = jnp.swapaxes(x.reshape(2, 128, 128), 1, 2)
    out_ref[...] = y.reshape(2, 1, 128, 128)


def _tc_transpose(tq):
    return pl.pallas_call(
        _trans_body,
        grid=(NCHUNK,),
        in_specs=[pl.BlockSpec((2, CTC, 8, 128), lambda i: (0, i, 0, 0))],
        out_specs=pl.BlockSpec((2, 1, 128, 128), lambda i: (0, i, 0, 0)),
        out_shape=jax.ShapeDtypeStruct((2, NCHUNK, 128, 128), jnp.float32),
    )(tq)


@functools.partial(
    pl.kernel,
    out_type=(
        jax.ShapeDtypeStruct((N_IDX, 8), jnp.float32),
        jax.ShapeDtypeStruct((N_IDX, 8), jnp.float32),
        jax.ShapeDtypeStruct((B,), jnp.float32),
    ),
    mesh=_sc_mesh,
    compiler_params=pltpu.CompilerParams(use_tc_tiling_on_sc=False,
                                        needs_layout_passes=False),
    scratch_types=[
        pltpu.VMEM((NSTREAM, K), jnp.int32),
        pltpu.VMEM((NSTREAM, K), jnp.int32),
        pltpu.VMEM((2, MB, 8), jnp.float32),
        pltpu.VMEM((2, MB, 8), jnp.float32),
        pltpu.VMEM((PER_W,), jnp.float32),
        pltpu.VMEM((SAMP_W,), jnp.float32),
        pltpu.SemaphoreType.DMA,
        pltpu.SemaphoreType.DMA,
        pltpu.SemaphoreType.DMA,
    ],
)
def _sc_gather(idx_hbm, rdx_hbm, half_hbm, first_hbm, lo_out, hi_out,
               fsum_out, idx_v, rdx_v, rows_lo, rows_hi, fst_v, fsum_v,
               rsem0, rsem1, fsem):
    wid = lax.axis_index("s") * NC + lax.axis_index("c")
    row0 = wid * NSTREAM
    base = wid * PER_W
    rsem = (rsem0, rsem1)
    pltpu.sync_copy(idx_hbm.at[pl.ds(row0, NSTREAM)], idx_v)
    pltpu.sync_copy(rdx_hbm.at[pl.ds(row0, NSTREAM)], rdx_v)

    def fire(m, p):
        for b in range(NB):
            j = m * NB + b
            pltpu.async_copy(half_hbm.at[0].at[rdx_v.at[j]],
                             rows_lo.at[p].at[pl.ds(b * K, K)], rsem[p])
            pltpu.async_copy(half_hbm.at[1].at[rdx_v.at[j]],
                             rows_hi.at[p].at[pl.ds(b * K, K)], rsem[p])
            pltpu.async_copy(first_hbm.at[idx_v.at[j]],
                             fst_v.at[pl.ds((m * NB + b) * K, K)], fsem)

    def drain(m, p):
        for b in range(NB):
            j = m * NB + b
            pltpu.make_async_copy(half_hbm.at[0].at[rdx_v.at[j]],
                                  rows_lo.at[p].at[pl.ds(b * K, K)],
                                  rsem[p]).wait()
            pltpu.make_async_copy(half_hbm.at[1].at[rdx_v.at[j]],
                                  rows_hi.at[p].at[pl.ds(b * K, K)],
                                  rsem[p]).wait()

    def write(m, p):
        pltpu.sync_copy(rows_lo.at[p], lo_out.at[pl.ds(base + m * MB, MB)])
        pltpu.sync_copy(rows_hi.at[p], hi_out.at[pl.ds(base + m * MB, MB)])

    fire(0, 0)

    def body(i, carry):
        mm = 2 * i
        fire(mm + 1, 1)
        drain(mm, 0)
        write(mm, 0)

        @pl.when(i < NMACRO // 2 - 1)
        def _():
            fire(mm + 2, 0)

        drain(mm + 1, 1)
        write(mm + 1, 1)
        return carry

    lax.fori_loop(0, NMACRO // 2, body, 0)

    def fdrain(j, carry):
        pltpu.make_async_copy(first_hbm.at[idx_v.at[j]],
                              fst_v.at[pl.ds(j * K, K)], fsem).wait()
        return carry

    lax.fori_loop(0, NSTREAM, fdrain, 0)

    iota16 = lax.iota(jnp.int32, 16)

    def fsum_body(g, carry):
        acc = jnp.zeros((16,), jnp.float32)
        for f in range(FIELD):
            ridx = (g * 16 + iota16) * FIELD + f
            acc = acc + plsc.load_gather(fst_v, [ridx])
        fsum_v[pl.ds(g * 16, 16)] = acc
        return carry

    lax.fori_loop(0, SAMP_W // 16, fsum_body, 0)
    pltpu.sync_copy(fsum_v, fsum_out.at[pl.ds(wid * SAMP_W, SAMP_W)])


def _mlp_body(lo_ref, hi_ref, x2_ref, fsum_ref, slo_ref, shi_ref,
              w1lo_ref, w1hi_ref, w1b_ref, b1_ref, g1_ref, be1_ref,
              w2_ref, b2_ref, g2_ref, be2_ref,
              w3_ref, b3_ref, g3_ref, be3_ref, bias_ref, out_ref):
    xlo = lo_ref[...]
    xhi = hi_ref[...]
    inv = 1.0 / jnp.sqrt(1.0 + EPS)
    sumvec = (jnp.dot(xlo, slo_ref[...], preferred_element_type=jnp.float32)
              + jnp.dot(xhi, shi_ref[...], preferred_element_type=jnp.float32))
    sq = jnp.sum(xlo * xlo, axis=1) + jnp.sum(xhi * xhi, axis=1)
    fm2 = 0.5 * (jnp.sum(sumvec * sumvec, axis=1) - sq)
    h = (jnp.dot(xlo, w1lo_ref[...], preferred_element_type=jnp.float32)
         + jnp.dot(xhi, w1hi_ref[...], preferred_element_type=jnp.float32)
         + jnp.dot(x2_ref[...], w1b_ref[...], preferred_element_type=jnp.float32)
         + b1_ref[...])
    h = jnp.maximum(h * (g1_ref[...] * inv) + be1_ref[...], 0.0)
    h = jnp.dot(h, w2_ref[...], preferred_element_type=jnp.float32) + b2_ref[...]
    h = jnp.maximum(h * (g2_ref[...] * inv) + be2_ref[...], 0.0)
    h = jnp.dot(h, w3_ref[...], preferred_element_type=jnp.float32) + b3_ref[...]
    h = jnp.maximum(h * (g3_ref[...] * inv) + be3_ref[...], 0.0)
    logit = fsum_ref[...] + fm2 + jnp.sum(h, axis=1) + bias_ref[0]
    out_ref[...] = jax.nn.sigmoid(logit)


def _mlp(xlo, xhi, x2, fsum, slo, shi, w1lo, w1hi, w1b, b1, g1, be1,
         w2, b2, g2, be2, w3, b3, g3, be3, bias):
    bs = 2048
    grid = (B // bs,)
    full = lambda shape: pl.BlockSpec(shape, lambda i: tuple(0 for _ in shape))
    return pl.pallas_call(
        _mlp_body,
        grid=grid,
        in_specs=[
            pl.BlockSpec((bs, DH), lambda i: (i, 0)),
            pl.BlockSpec((bs, DH), lambda i: (i, 0)),
            pl.BlockSpec((bs, NUMF), lambda i: (i, 0)),
            pl.BlockSpec((bs,), lambda i: (i,)),
            full(slo.shape), full(shi.shape), full(w1lo.shape),
            full(w1hi.shape), full(w1b.shape), full(b1.shape),
            full(g1.shape), full(be1.shape), full(w2.shape), full(b2.shape),
            full(g2.shape), full(be2.shape), full(w3.shape), full(b3.shape),
            full(g3.shape), full(be3.shape), full(bias.shape),
        ],
        out_specs=pl.BlockSpec((bs,), lambda i: (i,)),
        out_shape=jax.ShapeDtypeStruct((B,), jnp.float32),
    )(xlo, xhi, x2, fsum, slo, shi, w1lo, w1hi, w1b, b1, g1, be1,
      w2, b2, g2, be2, w3, b3, g3, be3, bias)


def kernel(train_x1, train_x2, fm_first_w, fm_second_w, bias,
           W1, b1, g1, be1, W2, b2, g2, be2, W3, b3, g3, be3):
    idx = train_x1.reshape(N_IDX // K, K)
    rdx = ((train_x1 >> 11) * 2048 + (train_x1 & 127) * 16
           + ((train_x1 >> 7) & 15)).reshape(N_IDX // K, K)
    tq = fm_second_w.T.reshape(2, 8, TCOL, 128).transpose(0, 2, 1, 3)
    half = _tc_transpose(tq).reshape(2, NROW8, 8)
    emb_lo, emb_hi, fsum = _sc_gather(idx, rdx, half, fm_first_w.T.reshape(V))
    xlo = emb_lo.reshape(B, DH)
    xhi = emb_hi.reshape(B, DH)
    eye = jnp.eye(EMB, dtype=jnp.float32)
    slo = jnp.tile(eye[:8], (FIELD, 1))
    shi = jnp.tile(eye[8:], (FIELD, 1))
    perm_lo = (jnp.arange(FIELD)[:, None] * EMB + jnp.arange(8)).reshape(-1)
    w1lo = W1[perm_lo]
    w1hi = W1[perm_lo + 8]
    return _mlp(xlo, xhi, train_x2, fsum, slo, shi, w1lo, w1hi, W1[D:],
                b1, g1, be1, W2, b2, g2, be2, W3, b3, g3, be3, bias)


# speedup vs baseline: 1.7079x; 1.7079x over previous
import functools

import jax
import jax.numpy as jnp
from jax import lax
from jax.experimental import pallas as pl
from jax.experimental.pallas import tpu as pltpu
from jax.experimental.pallas import tpu_sc as plsc

B = 16384
FIELD = 26
EMB = 16
V = 26 * 40000
D = FIELD * EMB
DH = D // 2
NUMF = 13
EPS = 1e-5

NC, NS = 2, 16
NW = NC * NS
N_IDX = B * FIELD
PER_W = N_IDX // NW
K = 128
NSTREAM = PER_W // K

NB = 4
MB = NB * K
NMACRO = NSTREAM // NB
SAMP_W = B // NW

TCOL = V // 128
GB = 8
CTC = 16 * GB
NCHUNK = (TCOL + CTC - 1) // CTC
NROW8 = NCHUNK * GB * 2048

_sc_mesh = plsc.VectorSubcoreMesh(core_axis_name="c", subcore_axis_name="s")


def _trans_body(in_ref, out_ref):
    x = in_ref[...]
    for g in range(GB):
        xg = x[:, g * 16:(g + 1) * 16].reshape(2, 128, 128)
        out_ref[:, g] = jnp.swapaxes(xg, 1, 2)


def _tc_transpose(tq):
    return pl.pallas_call(
        _trans_body,
        grid=(NCHUNK,),
        in_specs=[pl.BlockSpec((2, CTC, 8, 128), lambda i: (0, i, 0, 0))],
        out_specs=pl.BlockSpec((2, GB, 128, 128), lambda i: (0, i, 0, 0)),
        out_shape=jax.ShapeDtypeStruct((2, NCHUNK * GB, 128, 128), jnp.float32),
    )(tq)


@functools.partial(
    pl.kernel,
    out_type=(
        jax.ShapeDtypeStruct((N_IDX, 8), jnp.float32),
        jax.ShapeDtypeStruct((N_IDX, 8), jnp.float32),
        jax.ShapeDtypeStruct((B,), jnp.float32),
    ),
    mesh=_sc_mesh,
    compiler_params=pltpu.CompilerParams(use_tc_tiling_on_sc=False,
                                        needs_layout_passes=False),
    scratch_types=[
        pltpu.VMEM((NSTREAM, K), jnp.int32),
        pltpu.VMEM((NSTREAM, K), jnp.int32),
        pltpu.VMEM((2, MB, 8), jnp.float32),
        pltpu.VMEM((2, MB, 8), jnp.float32),
        pltpu.VMEM((PER_W,), jnp.float32),
        pltpu.VMEM((SAMP_W,), jnp.float32),
        pltpu.SemaphoreType.DMA,
        pltpu.SemaphoreType.DMA,
        pltpu.SemaphoreType.DMA,
    ],
)
def _sc_gather(idx_hbm, rdx_hbm, half_hbm, first_hbm, lo_out, hi_out,
               fsum_out, idx_v, rdx_v, rows_lo, rows_hi, fst_v, fsum_v,
               rsem0, rsem1, fsem):
    wid = lax.axis_index("s") * NC + lax.axis_index("c")
    row0 = wid * NSTREAM
    base = wid * PER_W
    rsem = (rsem0, rsem1)
    pltpu.sync_copy(idx_hbm.at[pl.ds(row0, NSTREAM)], idx_v)
    pltpu.sync_copy(rdx_hbm.at[pl.ds(row0, NSTREAM)], rdx_v)

    def fire(m, p):
        for b in range(NB):
            j = m * NB + b
            pltpu.async_copy(half_hbm.at[0].at[rdx_v.at[j]],
                             rows_lo.at[p].at[pl.ds(b * K, K)], rsem[p])
            pltpu.async_copy(half_hbm.at[1].at[rdx_v.at[j]],
                             rows_hi.at[p].at[pl.ds(b * K, K)], rsem[p])
            pltpu.async_copy(first_hbm.at[idx_v.at[j]],
                             fst_v.at[pl.ds((m * NB + b) * K, K)], fsem)

    def drain(m, p):
        for b in range(NB):
            j = m * NB + b
            pltpu.make_async_copy(half_hbm.at[0].at[rdx_v.at[j]],
                                  rows_lo.at[p].at[pl.ds(b * K, K)],
                                  rsem[p]).wait()
            pltpu.make_async_copy(half_hbm.at[1].at[rdx_v.at[j]],
                                  rows_hi.at[p].at[pl.ds(b * K, K)],
                                  rsem[p]).wait()

    def write(m, p):
        pltpu.sync_copy(rows_lo.at[p], lo_out.at[pl.ds(base + m * MB, MB)])
        pltpu.sync_copy(rows_hi.at[p], hi_out.at[pl.ds(base + m * MB, MB)])

    fire(0, 0)

    def body(i, carry):
        mm = 2 * i
        fire(mm + 1, 1)
        drain(mm, 0)
        write(mm, 0)

        @pl.when(i < NMACRO // 2 - 1)
        def _():
            fire(mm + 2, 0)

        drain(mm + 1, 1)
        write(mm + 1, 1)
        return carry

    lax.fori_loop(0, NMACRO // 2, body, 0)

    def fdrain(j, carry):
        pltpu.make_async_copy(first_hbm.at[idx_v.at[j]],
                              fst_v.at[pl.ds(j * K, K)], fsem).wait()
        return carry

    lax.fori_loop(0, NSTREAM, fdrain, 0)

    iota16 = lax.iota(jnp.int32, 16)

    def fsum_body(g, carry):
        acc = jnp.zeros((16,), jnp.float32)
        for f in range(FIELD):
            ridx = (g * 16 + iota16) * FIELD + f
            acc = acc + plsc.load_gather(fst_v, [ridx])
        fsum_v[pl.ds(g * 16, 16)] = acc
        return carry

    lax.fori_loop(0, SAMP_W // 16, fsum_body, 0)
    pltpu.sync_copy(fsum_v, fsum_out.at[pl.ds(wid * SAMP_W, SAMP_W)])


def _mlp_body(lo_ref, hi_ref, x2_ref, fsum_ref, slo_ref, shi_ref,
              w1lo_ref, w1hi_ref, w1b_ref, b1_ref, g1_ref, be1_ref,
              w2_ref, b2_ref, g2_ref, be2_ref,
              w3_ref, b3_ref, g3_ref, be3_ref, bias_ref, out_ref):
    xlo = lo_ref[...]
    xhi = hi_ref[...]
    inv = 1.0 / jnp.sqrt(1.0 + EPS)
    sumvec = (jnp.dot(xlo, slo_ref[...], preferred_element_type=jnp.float32)
              + jnp.dot(xhi, shi_ref[...], preferred_element_type=jnp.float32))
    sq = jnp.sum(xlo * xlo, axis=1) + jnp.sum(xhi * xhi, axis=1)
    fm2 = 0.5 * (jnp.sum(sumvec * sumvec, axis=1) - sq)
    h = (jnp.dot(xlo, w1lo_ref[...], preferred_element_type=jnp.float32)
         + jnp.dot(xhi, w1hi_ref[...], preferred_element_type=jnp.float32)
         + jnp.dot(x2_ref[...], w1b_ref[...], preferred_element_type=jnp.float32)
         + b1_ref[...])
    h = jnp.maximum(h * (g1_ref[...] * inv) + be1_ref[...], 0.0)
    h = jnp.dot(h, w2_ref[...], preferred_element_type=jnp.float32) + b2_ref[...]
    h = jnp.maximum(h * (g2_ref[...] * inv) + be2_ref[...], 0.0)
    h = jnp.dot(h, w3_ref[...], preferred_element_type=jnp.float32) + b3_ref[...]
    h = jnp.maximum(h * (g3_ref[...] * inv) + be3_ref[...], 0.0)
    logit = fsum_ref[...] + fm2 + jnp.sum(h, axis=1) + bias_ref[0]
    out_ref[...] = jax.nn.sigmoid(logit)


def _mlp(xlo, xhi, x2, fsum, slo, shi, w1lo, w1hi, w1b, b1, g1, be1,
         w2, b2, g2, be2, w3, b3, g3, be3, bias):
    bs = 2048
    grid = (B // bs,)
    full = lambda shape: pl.BlockSpec(shape, lambda i: tuple(0 for _ in shape))
    return pl.pallas_call(
        _mlp_body,
        grid=grid,
        in_specs=[
            pl.BlockSpec((bs, DH), lambda i: (i, 0)),
            pl.BlockSpec((bs, DH), lambda i: (i, 0)),
            pl.BlockSpec((bs, NUMF), lambda i: (i, 0)),
            pl.BlockSpec((bs,), lambda i: (i,)),
            full(slo.shape), full(shi.shape), full(w1lo.shape),
            full(w1hi.shape), full(w1b.shape), full(b1.shape),
            full(g1.shape), full(be1.shape), full(w2.shape), full(b2.shape),
            full(g2.shape), full(be2.shape), full(w3.shape), full(b3.shape),
            full(g3.shape), full(be3.shape), full(bias.shape),
        ],
        out_specs=pl.BlockSpec((bs,), lambda i: (i,)),
        out_shape=jax.ShapeDtypeStruct((B,), jnp.float32),
    )(xlo, xhi, x2, fsum, slo, shi, w1lo, w1hi, w1b, b1, g1, be1,
      w2, b2, g2, be2, w3, b3, g3, be3, bias)


def kernel(train_x1, train_x2, fm_first_w, fm_second_w, bias,
           W1, b1, g1, be1, W2, b2, g2, be2, W3, b3, g3, be3):
    idx = train_x1.reshape(N_IDX // K, K)
    rdx = ((train_x1 >> 11) * 2048 + (train_x1 & 127) * 16
           + ((train_x1 >> 7) & 15)).reshape(N_IDX // K, K)
    tq = fm_second_w.T.reshape(2, 8, TCOL, 128).transpose(0, 2, 1, 3)
    half = _tc_transpose(tq).reshape(2, NROW8, 8)
    emb_lo, emb_hi, fsum = _sc_gather(idx, rdx, half, fm_first_w.T.reshape(V))
    xlo = emb_lo.reshape(B, DH)
    xhi = emb_hi.reshape(B, DH)
    eye = jnp.eye(EMB, dtype=jnp.float32)
    slo = jnp.tile(eye[:8], (FIELD, 1))
    shi = jnp.tile(eye[8:], (FIELD, 1))
    perm_lo = (jnp.arange(FIELD)[:, None] * EMB + jnp.arange(8)).reshape(-1)
    w1lo = W1[perm_lo]
    w1hi = W1[perm_lo + 8]
    return _mlp(xlo, xhi, train_x2, fsum, slo, shi, w1lo, w1hi, W1[D:],
                b1, g1, be1, W2, b2, g2, be2, W3, b3, g3, be3, bias)


# speedup vs baseline: 1.7195x; 1.0068x over previous
import functools

import jax
import jax.numpy as jnp
from jax import lax
from jax.experimental import pallas as pl
from jax.experimental.pallas import tpu as pltpu
from jax.experimental.pallas import tpu_sc as plsc

B = 16384
FIELD = 26
EMB = 16
V = 26 * 40000
D = FIELD * EMB
DH = D // 2
NUMF = 13
EPS = 1e-5

NC, NS = 2, 16
NW = NC * NS
N_IDX = B * FIELD
PER_W = N_IDX // NW
K = 128
NSTREAM = PER_W // K

NB = 4
MB = NB * K
NMACRO = NSTREAM // NB
SAMP_W = B // NW

TCOL = V // 128
GB = 8
CTC = 16 * GB
NCHUNK = (TCOL + CTC - 1) // CTC
NROW8 = NCHUNK * GB * 2048

_sc_mesh = plsc.VectorSubcoreMesh(core_axis_name="c", subcore_axis_name="s")


def _trans_body(in_ref, out_ref):
    x = in_ref[...]
    for g in range(GB):
        xg = x[:, g * 16:(g + 1) * 16].reshape(2, 128, 128)
        out_ref[:, g] = jnp.swapaxes(xg, 1, 2)


def _tc_transpose(tq):
    return pl.pallas_call(
        _trans_body,
        grid=(NCHUNK,),
        in_specs=[pl.BlockSpec((2, CTC, 8, 128), lambda i: (0, i, 0, 0))],
        out_specs=pl.BlockSpec((2, GB, 128, 128), lambda i: (0, i, 0, 0)),
        out_shape=jax.ShapeDtypeStruct((2, NCHUNK * GB, 128, 128), jnp.float32),
    )(tq)


@functools.partial(
    pl.kernel,
    out_type=(
        jax.ShapeDtypeStruct((N_IDX, 8), jnp.float32),
        jax.ShapeDtypeStruct((N_IDX, 8), jnp.float32),
        jax.ShapeDtypeStruct((B,), jnp.float32),
    ),
    mesh=_sc_mesh,
    compiler_params=pltpu.CompilerParams(use_tc_tiling_on_sc=False,
                                        needs_layout_passes=False),
    scratch_types=[
        pltpu.VMEM((NSTREAM, K), jnp.int32),
        pltpu.VMEM((NSTREAM, K), jnp.int32),
        pltpu.VMEM((2, MB, 8), jnp.float32),
        pltpu.VMEM((2, MB, 8), jnp.float32),
        pltpu.VMEM((PER_W,), jnp.float32),
        pltpu.VMEM((SAMP_W,), jnp.float32),
        pltpu.SemaphoreType.DMA,
        pltpu.SemaphoreType.DMA,
        pltpu.SemaphoreType.DMA,
    ],
)
def _sc_gather(idx_hbm, rdx_hbm, half_hbm, first_hbm, lo_out, hi_out,
               fsum_out, idx_v, rdx_v, rows_lo, rows_hi, fst_v, fsum_v,
               rsem0, rsem1, fsem):
    wid = lax.axis_index("s") * NC + lax.axis_index("c")
    row0 = wid * NSTREAM
    base = wid * PER_W
    rsem = (rsem0, rsem1)
    pltpu.sync_copy(idx_hbm.at[pl.ds(row0, NSTREAM)], idx_v)
    pltpu.sync_copy(rdx_hbm.at[pl.ds(row0, NSTREAM)], rdx_v)

    def fire(m, p):
        for b in range(NB):
            j = m * NB + b
            pltpu.async_copy(half_hbm.at[0].at[rdx_v.at[j]],
                             rows_lo.at[p].at[pl.ds(b * K, K)], rsem[p])
            pltpu.async_copy(half_hbm.at[1].at[rdx_v.at[j]],
                             rows_hi.at[p].at[pl.ds(b * K, K)], rsem[p])
            pltpu.async_copy(first_hbm.at[idx_v.at[j]],
                             fst_v.at[pl.ds((m * NB + b) * K, K)], fsem)

    def drain(m, p):
        for b in range(NB):
            j = m * NB + b
            pltpu.make_async_copy(half_hbm.at[0].at[rdx_v.at[j]],
                                  rows_lo.at[p].at[pl.ds(b * K, K)],
                                  rsem[p]).wait()
            pltpu.make_async_copy(half_hbm.at[1].at[rdx_v.at[j]],
                                  rows_hi.at[p].at[pl.ds(b * K, K)],
                                  rsem[p]).wait()

    def write(m, p):
        pltpu.sync_copy(rows_lo.at[p], lo_out.at[pl.ds(base + m * MB, MB)])
        pltpu.sync_copy(rows_hi.at[p], hi_out.at[pl.ds(base + m * MB, MB)])

    fire(0, 0)

    def body(i, carry):
        mm = 2 * i
        fire(mm + 1, 1)
        drain(mm, 0)
        write(mm, 0)

        @pl.when(i < NMACRO // 2 - 1)
        def _():
            fire(mm + 2, 0)

        drain(mm + 1, 1)
        write(mm + 1, 1)
        return carry

    lax.fori_loop(0, NMACRO // 2, body, 0)

    def fdrain(j, carry):
        pltpu.make_async_copy(first_hbm.at[idx_v.at[j]],
                              fst_v.at[pl.ds(j * K, K)], fsem).wait()
        return carry

    lax.fori_loop(0, NSTREAM, fdrain, 0)

    iota16 = lax.iota(jnp.int32, 16)

    def fsum_body(g, carry):
        acc = jnp.zeros((16,), jnp.float32)
        for f in range(FIELD):
            ridx = (g * 16 + iota16) * FIELD + f
            acc = acc + plsc.load_gather(fst_v, [ridx])
        fsum_v[pl.ds(g * 16, 16)] = acc
        return carry

    lax.fori_loop(0, SAMP_W // 16, fsum_body, 0)
    pltpu.sync_copy(fsum_v, fsum_out.at[pl.ds(wid * SAMP_W, SAMP_W)])


def _mlp_body(lo_ref, hi_ref, x2_ref, fsum_ref, slo_ref, shi_ref,
              w1lo_ref, w1hi_ref, w1b_ref, b1_ref, g1_ref, be1_ref,
              w2_ref, b2_ref, g2_ref, be2_ref,
              w3_ref, b3_ref, g3_ref, be3_ref, bias_ref, out_ref):
    xlo = lo_ref[...]
    xhi = hi_ref[...]
    inv = 1.0 / jnp.sqrt(1.0 + EPS)
    sumvec = (jnp.dot(xlo, slo_ref[...], preferred_element_type=jnp.float32)
              + jnp.dot(xhi, shi_ref[...], preferred_element_type=jnp.float32))
    sq = jnp.sum(xlo * xlo, axis=1) + jnp.sum(xhi * xhi, axis=1)
    fm2 = 0.5 * (jnp.sum(sumvec * sumvec, axis=1) - sq)
    h = (jnp.dot(xlo, w1lo_ref[...], preferred_element_type=jnp.float32)
         + jnp.dot(xhi, w1hi_ref[...], preferred_element_type=jnp.float32)
         + jnp.dot(x2_ref[...], w1b_ref[...], preferred_element_type=jnp.float32)
         + b1_ref[...])
    h = jnp.maximum(h * (g1_ref[...] * inv) + be1_ref[...], 0.0)
    h = jnp.dot(h, w2_ref[...], preferred_element_type=jnp.float32) + b2_ref[...]
    h = jnp.maximum(h * (g2_ref[...] * inv) + be2_ref[...], 0.0)
    h = jnp.dot(h, w3_ref[...], preferred_element_type=jnp.float32) + b3_ref[...]
    h = jnp.maximum(h * (g3_ref[...] * inv) + be3_ref[...], 0.0)
    logit = fsum_ref[...] + fm2 + jnp.sum(h, axis=1) + bias_ref[0]
    out_ref[...] = jax.nn.sigmoid(logit)


def _mlp(xlo, xhi, x2, fsum, slo, shi, w1lo, w1hi, w1b, b1, g1, be1,
         w2, b2, g2, be2, w3, b3, g3, be3, bias):
    bs = 4096
    grid = (B // bs,)
    full = lambda shape: pl.BlockSpec(shape, lambda i: tuple(0 for _ in shape))
    return pl.pallas_call(
        _mlp_body,
        grid=grid,
        in_specs=[
            pl.BlockSpec((bs, DH), lambda i: (i, 0)),
            pl.BlockSpec((bs, DH), lambda i: (i, 0)),
            pl.BlockSpec((bs, NUMF), lambda i: (i, 0)),
            pl.BlockSpec((bs,), lambda i: (i,)),
            full(slo.shape), full(shi.shape), full(w1lo.shape),
            full(w1hi.shape), full(w1b.shape), full(b1.shape),
            full(g1.shape), full(be1.shape), full(w2.shape), full(b2.shape),
            full(g2.shape), full(be2.shape), full(w3.shape), full(b3.shape),
            full(g3.shape), full(be3.shape), full(bias.shape),
        ],
        out_specs=pl.BlockSpec((bs,), lambda i: (i,)),
        out_shape=jax.ShapeDtypeStruct((B,), jnp.float32),
    )(xlo, xhi, x2, fsum, slo, shi, w1lo, w1hi, w1b, b1, g1, be1,
      w2, b2, g2, be2, w3, b3, g3, be3, bias)


def kernel(train_x1, train_x2, fm_first_w, fm_second_w, bias,
           W1, b1, g1, be1, W2, b2, g2, be2, W3, b3, g3, be3):
    idx = train_x1.reshape(N_IDX // K, K)
    rdx = (idx >> 11) * 2048 + (idx & 127) * 16 + ((idx >> 7) & 15)
    tq = fm_second_w.T.reshape(2, 8, TCOL, 128).transpose(0, 2, 1, 3)
    half = _tc_transpose(tq).reshape(2, NROW8, 8)
    emb_lo, emb_hi, fsum = _sc_gather(idx, rdx, half, fm_first_w.T.reshape(V))
    xlo = emb_lo.reshape(B, DH)
    xhi = emb_hi.reshape(B, DH)
    eye = jnp.eye(EMB, dtype=jnp.float32)
    slo = jnp.tile(eye[:8], (FIELD, 1))
    shi = jnp.tile(eye[8:], (FIELD, 1))
    perm_lo = (jnp.arange(FIELD)[:, None] * EMB + jnp.arange(8)).reshape(-1)
    w1lo = W1[perm_lo]
    w1hi = W1[perm_lo + 8]
    return _mlp(xlo, xhi, train_x2, fsum, slo, shi, w1lo, w1hi, W1[D:],
                b1, g1, be1, W2, b2, g2, be2, W3, b3, g3, be3, bias)
